# baseline (speedup 1.0000x reference)
import functools
import math

import jax
import jax.numpy as jnp
from jax.experimental import pallas as pl
from jax.experimental.pallas import tpu as pltpu

D_MODEL_K = 4096
MAX_NORM_K = math.sqrt(D_MODEL_K)
CHUNK_ROWS = 512
N_IN = 4
N_OUT = 3


def _pipeline_kernel(ts_ref, x_hbm, tbl_hbm, o_hbm, ibuf, obuf, emb_ref,
                     in_sems, out_sems, row_sem, *, n_chunks, chunks_per_b,
                     n_batch):
    for b in range(n_batch):
        pltpu.make_async_copy(
            tbl_hbm.at[pl.ds(ts_ref[b], 1), :], emb_ref.at[pl.ds(b, 1), :],
            row_sem,
        ).start()

    def copy_in(c, slot):
        return pltpu.make_async_copy(
            x_hbm.at[pl.ds(c * CHUNK_ROWS, CHUNK_ROWS), :],
            ibuf.at[slot],
            in_sems.at[slot],
        )

    def copy_out(c, slot):
        return pltpu.make_async_copy(
            obuf.at[slot],
            o_hbm.at[pl.ds(c * CHUNK_ROWS, CHUNK_ROWS), :],
            out_sems.at[slot],
        )

    for s in range(min(N_IN, n_chunks)):
        copy_in(s, s).start()

    for b in range(n_batch):
        pltpu.make_async_copy(
            tbl_hbm.at[pl.ds(ts_ref[b], 1), :], emb_ref.at[pl.ds(b, 1), :],
            row_sem,
        ).wait()
    rows = emb_ref[...]
    norms = jnp.sqrt(jnp.sum(rows * rows, axis=-1, keepdims=True))
    emb_ref[...] = rows * jnp.where(norms > MAX_NORM_K,
                                    MAX_NORM_K / (norms + 1e-7), 1.0)

    for c in range(n_chunks):
        si = c % N_IN
        so = c % N_OUT
        b = c // chunks_per_b
        copy_in(c, si).wait()
        if c >= N_OUT:
            copy_out(c - N_OUT, so).wait()
        obuf[so] = ibuf[si] + emb_ref[pl.ds(b, 1), :]
        copy_out(c, so).start()
        nxt = c + N_IN
        if nxt < n_chunks:
            copy_in(nxt, si).start()

    for c in range(max(0, n_chunks - N_OUT), n_chunks):
        copy_out(c, c % N_OUT).wait()


def kernel(x, timesteps, table):
    B, S, D = x.shape
    x2 = x.reshape(B * S, D)
    n_chunks = (B * S) // CHUNK_ROWS
    chunks_per_b = S // CHUNK_ROWS
    body = functools.partial(_pipeline_kernel, n_chunks=n_chunks,
                             chunks_per_b=chunks_per_b, n_batch=B)
    out = pl.pallas_call(
        body,
        grid_spec=pltpu.PrefetchScalarGridSpec(
            num_scalar_prefetch=1,
            grid=(1,),
            in_specs=[
                pl.BlockSpec(memory_space=pl.ANY),
                pl.BlockSpec(memory_space=pl.ANY),
            ],
            out_specs=pl.BlockSpec(memory_space=pl.ANY),
            scratch_shapes=[
                pltpu.VMEM((N_IN, CHUNK_ROWS, D), x.dtype),
                pltpu.VMEM((N_OUT, CHUNK_ROWS, D), x.dtype),
                pltpu.VMEM((B, D), x.dtype),
                pltpu.SemaphoreType.DMA((N_IN,)),
                pltpu.SemaphoreType.DMA((N_OUT,)),
                pltpu.SemaphoreType.DMA,
            ],
        ),
        out_shape=jax.ShapeDtypeStruct(x2.shape, x.dtype),
    )(timesteps, x2, table)
    return out.reshape(B, S, D)


# speedup vs baseline: 1.0007x; 1.0007x over previous
import functools
import math

import jax
import jax.numpy as jnp
from jax.experimental import pallas as pl
from jax.experimental.pallas import tpu as pltpu

D_MODEL_K = 4096
MAX_NORM_K = math.sqrt(D_MODEL_K)
CHUNK = 1024
NBUF = 3
NSPLIT = 2


def _pipeline_kernel(ts_ref, x_hbm, tbl_hbm, o_hbm, buf, emb_ref,
                     in_sems, out_sems, row_sem, *, n_chunks, chunks_per_b,
                     n_batch):
    for b in range(n_batch):
        pltpu.make_async_copy(
            tbl_hbm.at[pl.ds(ts_ref[b], 1), :], emb_ref.at[pl.ds(b, 1), :],
            row_sem,
        ).start()

    sub = CHUNK // NSPLIT

    def copies_in(c, slot):
        return [
            pltpu.make_async_copy(
                x_hbm.at[pl.ds(c * CHUNK + k * sub, sub), :],
                buf.at[slot, pl.ds(k * sub, sub), :],
                in_sems.at[slot],
            )
            for k in range(NSPLIT)
        ]

    def copies_out(c, slot):
        return [
            pltpu.make_async_copy(
                buf.at[slot, pl.ds(k * sub, sub), :],
                o_hbm.at[pl.ds(c * CHUNK + k * sub, sub), :],
                out_sems.at[slot],
            )
            for k in range(NSPLIT)
        ]

    def start(cps):
        for cp in cps:
            cp.start()

    def wait(cps):
        for cp in cps:
            cp.wait()

    for s in range(min(NBUF, n_chunks)):
        start(copies_in(s, s))

    for b in range(n_batch):
        pltpu.make_async_copy(
            tbl_hbm.at[pl.ds(ts_ref[b], 1), :], emb_ref.at[pl.ds(b, 1), :],
            row_sem,
        ).wait()
    rows = emb_ref[...]
    norms = jnp.sqrt(jnp.sum(rows * rows, axis=-1, keepdims=True))
    emb_ref[...] = rows * jnp.where(norms > MAX_NORM_K,
                                    MAX_NORM_K / (norms + 1e-7), 1.0)

    for c in range(n_chunks):
        slot = c % NBUF
        b = c // chunks_per_b
        wait(copies_in(c, slot))
        buf[slot] += emb_ref[pl.ds(b, 1), :]
        start(copies_out(c, slot))
        nxt = c + NBUF
        if nxt < n_chunks:
            wait(copies_out(c, slot))
            start(copies_in(nxt, slot))

    for c in range(max(0, n_chunks - NBUF), n_chunks):
        wait(copies_out(c, c % NBUF))


def kernel(x, timesteps, table):
    B, S, D = x.shape
    x2 = x.reshape(B * S, D)
    n_chunks = (B * S) // CHUNK
    chunks_per_b = S // CHUNK
    body = functools.partial(_pipeline_kernel, n_chunks=n_chunks,
                             chunks_per_b=chunks_per_b, n_batch=B)
    out = pl.pallas_call(
        body,
        grid_spec=pltpu.PrefetchScalarGridSpec(
            num_scalar_prefetch=1,
            grid=(1,),
            in_specs=[
                pl.BlockSpec(memory_space=pl.ANY),
                pl.BlockSpec(memory_space=pl.ANY),
            ],
            out_specs=pl.BlockSpec(memory_space=pl.ANY),
            scratch_shapes=[
                pltpu.VMEM((NBUF, CHUNK, D), x.dtype),
                pltpu.VMEM((B, D), x.dtype),
                pltpu.SemaphoreType.DMA((NBUF,)),
                pltpu.SemaphoreType.DMA((NBUF,)),
                pltpu.SemaphoreType.DMA,
            ],
        ),
        out_shape=jax.ShapeDtypeStruct(x2.shape, x.dtype),
    )(timesteps, x2, table)
    return out.reshape(B, S, D)
